# baseline (speedup 1.0000x reference)
import functools
import math

import jax
import jax.numpy as jnp
from jax import lax
from jax.experimental import pallas as pl
from jax.experimental.pallas import tpu as pltpu
from jax.experimental.pallas import tpu_sc as plsc

_L = 16
_NC = 2
_NS = 16


@functools.partial(jax.jit, static_argnames=("h", "w", "d_model"))
def _pos_embed_sc(row_flat, col_flat, h, w, d_model):
    nw = _NC * _NS
    rpw = d_model // nw

    mesh = plsc.VectorSubcoreMesh(
        core_axis_name="c", subcore_axis_name="s",
        num_cores=_NC, num_subcores=_NS,
    )

    @functools.partial(
        pl.kernel,
        out_type=jax.ShapeDtypeStruct((d_model, h * w), jnp.float32),
        mesh=mesh,
        scratch_types=[
            pltpu.VMEM((h, rpw), jnp.float32),
            pltpu.VMEM((w, rpw), jnp.float32),
            pltpu.VMEM((rpw, h * w), jnp.float32),
            pltpu.SemaphoreType.DMA,
        ],
        compiler_params=pltpu.CompilerParams(needs_layout_passes=False),
    )
    def body(col_hbm, row_hbm, out_hbm, colv, rowv, outv, sem):
        wid = lax.axis_index("s") * _NC + lax.axis_index("c")
        d0 = wid * rpw
        copies = []
        for i in range(h):
            copies.append(pltpu.async_copy(
                col_hbm.at[pl.ds(i * d_model + d0, rpw)], colv.at[i], sem))
            copies.append(pltpu.async_copy(
                row_hbm.at[pl.ds(i * d_model + d0, rpw)], rowv.at[i], sem))
        for c in copies:
            c.wait()

        lanes = lax.iota(jnp.int32, _L)
        for r in range(rpw):
            rsplat = jnp.full((_L,), r, jnp.int32)
            bs = [
                plsc.load_gather(rowv, [lanes + q * _L, rsplat])
                for q in range(w // _L)
            ]

            def blk_body(ib, _, r=r, bs=bs, rsplat=rsplat):
                av = plsc.load_gather(colv, [lanes + ib * _L, rsplat])
                for li in range(_L):
                    a = av[li]
                    base = (ib * _L + li) * w
                    for q in range(w // _L):
                        outv[r, pl.ds(base + q * _L, _L)] = a + bs[q]
                return 0

            lax.fori_loop(0, h // _L, blk_body, 0)
        pltpu.sync_copy(outv, out_hbm.at[pl.ds(d0, rpw)])

    return body(col_flat, row_flat)


def kernel(patch, row_embed, col_embed):
    hw = patch.shape[0]
    h = int(math.isqrt(hw))
    w = h
    d_model = row_embed.shape[1]
    out2d = _pos_embed_sc(
        row_embed.reshape(-1), col_embed.reshape(-1), h, w, d_model)
    return out2d.reshape(1, d_model, h, w)
